# baseline (speedup 1.0000x reference)
import functools

import jax
import jax.numpy as jnp
from jax import lax
from jax.experimental import pallas as pl
from jax.experimental.pallas import tpu as pltpu
from jax.experimental.pallas import tpu_sc as plsc

N = 10000
E = 320000
NB = 2 * N
D = 64
NC, NS = 2, 16
NW = NC * NS
CH = 128
TRASH = NB
ACC_ROWS = 20480
RZ = ACC_ROWS // NS
EA = 2 * E
NBUF = 4
CPW = 160
NCHUNK = NW * CPW
EPAD = NCHUNK * CH
ROUNDS = CPW // NBUF

_mesh = plsc.VectorSubcoreMesh(core_axis_name="c", subcore_axis_name="s")


@functools.partial(
    pl.kernel,
    out_type=jax.ShapeDtypeStruct((NC, ACC_ROWS, D), jnp.float32),
    mesh=_mesh,
    scratch_types=[
        pltpu.VMEM((NBUF, CH), jnp.int32),
        pltpu.VMEM((NBUF, CH), jnp.int32),
        pltpu.VMEM((NBUF, CH, D), jnp.float32),
        pltpu.VMEM_SHARED((ACC_ROWS, D), jnp.float32),
        pltpu.SemaphoreType.DMA((NBUF,)),
        pltpu.SemaphoreType.DMA((NBUF,)),
        pltpu.SemaphoreType.DMA((NBUF,)),
    ],
    compiler_params=pltpu.CompilerParams(use_tc_tiling_on_sc=False),
)
def _sc_agg(table_h, src_h, dst_h, zeros_h, out_h, sidx, didx, rows, acc,
            isem, gsem, ssem):
    c = lax.axis_index("c")
    s = lax.axis_index("s")
    wid = c * NS + s
    base = wid * CPW
    pltpu.sync_copy(zeros_h.at[pl.ds(s * RZ, RZ)], acc.at[pl.ds(s * RZ, RZ)])
    plsc.subcore_barrier()

    def idx_load(k, b):
        pltpu.async_copy(src_h.at[base + k], sidx.at[b], isem.at[b])
        pltpu.async_copy(dst_h.at[base + k], didx.at[b], isem.at[b])

    def idx_wait(k, b):
        pltpu.make_async_copy(src_h.at[base + k], sidx.at[b], isem.at[b]).wait()
        pltpu.make_async_copy(dst_h.at[base + k], didx.at[b], isem.at[b]).wait()

    for b in range(NBUF):
        idx_load(b, b)

    def round_(r, carry):
        for b in range(NBUF):
            k = r * NBUF + b
            idx_wait(k, b)
            pltpu.async_copy(table_h.at[sidx.at[b]], rows.at[b], gsem.at[b])
        for b in range(NBUF):
            k = r * NBUF + b
            pltpu.make_async_copy(
                table_h.at[sidx.at[b]], rows.at[b], gsem.at[b]).wait()
            pltpu.async_copy(rows.at[b], acc.at[didx.at[b]], ssem.at[b],
                             add=True)
        for b in range(NBUF):
            k = r * NBUF + b
            pltpu.make_async_copy(
                rows.at[b], acc.at[didx.at[b]], ssem.at[b]).wait()

            def refire(b=b, k=k):
                idx_load(k + NBUF, b)
            pl.when(r < ROUNDS - 1)(refire)
        return carry

    lax.fori_loop(0, ROUNDS, round_, 0)
    plsc.subcore_barrier()
    pltpu.sync_copy(acc.at[pl.ds(s * RZ, RZ)], out_h.at[c].at[pl.ds(s * RZ, RZ)])


@functools.partial(
    pl.kernel,
    out_type=jax.ShapeDtypeStruct((NC, ACC_ROWS, 16), jnp.float32),
    mesh=_mesh,
    scratch_types=[
        pltpu.VMEM((CPW, CH), jnp.int32),
        pltpu.VMEM((CH, 16), jnp.float32),
        pltpu.VMEM_SHARED((ACC_ROWS, 16), jnp.float32),
        pltpu.SemaphoreType.DMA((NBUF,)),
    ],
    compiler_params=pltpu.CompilerParams(use_tc_tiling_on_sc=False),
)
def _sc_deg(dst_h, zeros_h, ones_h, out_h, didx, ones_v, acc, ssem):
    c = lax.axis_index("c")
    s = lax.axis_index("s")
    wid = c * NS + s
    pltpu.sync_copy(dst_h.at[pl.ds(wid * CPW, CPW)], didx)
    pltpu.sync_copy(zeros_h.at[pl.ds(s * RZ, RZ)], acc.at[pl.ds(s * RZ, RZ)])
    pltpu.sync_copy(ones_h, ones_v)
    plsc.subcore_barrier()

    def round_(r, carry):
        for b in range(NBUF):
            k = r * NBUF + b
            pltpu.async_copy(ones_v, acc.at[didx.at[k]], ssem.at[b], add=True)
        for b in range(NBUF):
            k = r * NBUF + b
            pltpu.make_async_copy(ones_v, acc.at[didx.at[k]], ssem.at[b]).wait()
        return carry

    lax.fori_loop(0, ROUNDS, round_, 0)
    plsc.subcore_barrier()
    pltpu.sync_copy(acc.at[pl.ds(s * RZ, RZ)], out_h.at[c].at[pl.ds(s * RZ, RZ)])


_BR = 1000
_G = NB // _BR


def _dinv_body(p0_ref, p1_ref, o_ref):
    deg = p0_ref[0, :, 0:1] + p1_ref[0, :, 0:1] + 1.0
    o_ref[...] = jnp.broadcast_to(lax.rsqrt(deg), o_ref.shape)


def _dinv(parts16):
    return pl.pallas_call(
        _dinv_body,
        grid=(_G,),
        in_specs=[
            pl.BlockSpec((1, _BR, 16), lambda i: (0, i, 0)),
            pl.BlockSpec((1, _BR, 16), lambda i: (1, i, 0)),
        ],
        out_specs=pl.BlockSpec((_BR, D), lambda i: (i, 0)),
        out_shape=jax.ShapeDtypeStruct((NB, D), jnp.float32),
    )(parts16, parts16)


def _mm1_body(x_ref, w_ref, dinv_ref, o_ref):
    h = jnp.dot(x_ref[...], w_ref[0], preferred_element_type=jnp.float32)
    o_ref[...] = h * dinv_ref[...]


def _mm1(x, w, dinv):
    fin = x.shape[1]
    return pl.pallas_call(
        _mm1_body,
        grid=(_G,),
        in_specs=[
            pl.BlockSpec((_BR, fin), lambda i: (i, 0)),
            pl.BlockSpec((1, fin, D), lambda i: (i // (_G // 2), 0, 0)),
            pl.BlockSpec((_BR, D), lambda i: (i, 0)),
        ],
        out_specs=pl.BlockSpec((_BR, D), lambda i: (i, 0)),
        out_shape=jax.ShapeDtypeStruct((NB, D), jnp.float32),
    )(x, w, dinv)


def _mid_body(p0_ref, p1_ref, hs_ref, dinv_ref, b_ref, w_ref, o_ref):
    b = b_ref[0, 0:1, :]
    x = dinv_ref[...] * (p0_ref[0] + p1_ref[0] + hs_ref[...]) + b
    x = jnp.maximum(x, 0.0)
    h = jnp.dot(x, w_ref[0], preferred_element_type=jnp.float32)
    o_ref[...] = h * dinv_ref[...]


def _mid(parts, hs, dinv, b, w):
    return pl.pallas_call(
        _mid_body,
        grid=(_G,),
        in_specs=[
            pl.BlockSpec((1, _BR, D), lambda i: (0, i, 0)),
            pl.BlockSpec((1, _BR, D), lambda i: (1, i, 0)),
            pl.BlockSpec((_BR, D), lambda i: (i, 0)),
            pl.BlockSpec((_BR, D), lambda i: (i, 0)),
            pl.BlockSpec((1, 8, D), lambda i: (i // (_G // 2), 0, 0)),
            pl.BlockSpec((1, D, D), lambda i: (i // (_G // 2), 0, 0)),
        ],
        out_specs=pl.BlockSpec((_BR, D), lambda i: (i, 0)),
        out_shape=jax.ShapeDtypeStruct((NB, D), jnp.float32),
    )(parts, parts, hs, dinv, b, w)


def _fin_body(p0_ref, p1_ref, hs_ref, dinv_ref, b_ref, o_ref):
    b = b_ref[0, 0:1, :]
    o_ref[...] = dinv_ref[...] * (p0_ref[0] + p1_ref[0] + hs_ref[...]) + b


def _fin(parts, hs, dinv, b):
    return pl.pallas_call(
        _fin_body,
        grid=(_G,),
        in_specs=[
            pl.BlockSpec((1, _BR, D), lambda i: (0, i, 0)),
            pl.BlockSpec((1, _BR, D), lambda i: (1, i, 0)),
            pl.BlockSpec((_BR, D), lambda i: (i, 0)),
            pl.BlockSpec((_BR, D), lambda i: (i, 0)),
            pl.BlockSpec((1, 8, D), lambda i: (i // (_G // 2), 0, 0)),
        ],
        out_specs=pl.BlockSpec((_BR, D), lambda i: (i, 0)),
        out_shape=jax.ShapeDtypeStruct((NB, D), jnp.float32),
    )(parts, parts, hs, dinv, b)


def kernel(x_data_matrix, x_edge_index, y_data_matrix, y_edge_index,
           Wx1, bx1, Wx2, bx2, Wx3, bx3, Wx4, bx4, Wx5, bx5,
           Wy1, by1, Wy2, by2, Wy3, by3, Wy4, by4, Wy5, by5):
    pad = EPAD - EA
    srcs = jnp.concatenate([
        x_edge_index[0], y_edge_index[0] + N,
        jnp.zeros((pad,), jnp.int32)])
    dsts = jnp.concatenate([
        x_edge_index[1], y_edge_index[1] + N,
        jnp.full((pad,), TRASH, jnp.int32)])
    src2d = srcs.reshape(NCHUNK, CH)
    dst2d = dsts.reshape(NCHUNK, CH)

    zeros64 = jnp.zeros((ACC_ROWS, D), jnp.float32)
    zeros16 = jnp.zeros((ACC_ROWS, 16), jnp.float32)
    ones = jnp.ones((CH, 16), jnp.float32)

    ws = [(jnp.stack([Wx1, Wy1]), jnp.stack([bx1, by1])),
          (jnp.stack([Wx2, Wy2]), jnp.stack([bx2, by2])),
          (jnp.stack([Wx3, Wy3]), jnp.stack([bx3, by3])),
          (jnp.stack([Wx4, Wy4]), jnp.stack([bx4, by4])),
          (jnp.stack([Wx5, Wy5]), jnp.stack([bx5, by5]))]
    bs = [jnp.broadcast_to(b[:, None, :], (2, 8, D)) for _, b in ws]

    xall = jnp.concatenate([x_data_matrix, y_data_matrix], axis=0)

    degp = _sc_deg(dst2d, zeros16, ones)
    dinv = _dinv(degp)

    hs = _mm1(xall, ws[0][0], dinv)
    out = None
    for li in range(5):
        parts = _sc_agg(hs, src2d, dst2d, zeros64)
        if li < 4:
            hs = _mid(parts, hs, dinv, bs[li], ws[li + 1][0])
        else:
            out = _fin(parts, hs, dinv, bs[4])
    return out[:N], out[N:]


# speedup vs baseline: 2.0741x; 2.0741x over previous
import functools

import jax
import jax.numpy as jnp
from jax import lax
from jax.experimental import pallas as pl
from jax.experimental.pallas import tpu as pltpu
from jax.experimental.pallas import tpu_sc as plsc

N = 10000
E = 320000
D = 64
NC, NS = 2, 16
CH = 128
TRASH = N
ACC_ROWS = 10240
RZ = ACC_ROWS // NS
NBUF = 4
CPW = 160
CPB = NS * CPW
EPADB = CPB * CH
ROUNDS = CPW // NBUF
TSTR = N // NS

_mesh = plsc.VectorSubcoreMesh(core_axis_name="c", subcore_axis_name="s")


@functools.partial(
    pl.kernel,
    out_type=jax.ShapeDtypeStruct((NC, ACC_ROWS, D), jnp.float32),
    mesh=_mesh,
    scratch_types=[
        pltpu.VMEM((NBUF, CH), jnp.int32),
        pltpu.VMEM((NBUF, CH), jnp.int32),
        pltpu.VMEM((NBUF, CH, D), jnp.float32),
        pltpu.VMEM_SHARED((ACC_ROWS, D), jnp.float32),
        pltpu.VMEM_SHARED((ACC_ROWS, D), jnp.float32),
        pltpu.SemaphoreType.DMA((NBUF,)),
        pltpu.SemaphoreType.DMA((NBUF,)),
        pltpu.SemaphoreType.DMA((NBUF,)),
    ],
    compiler_params=pltpu.CompilerParams(use_tc_tiling_on_sc=False),
)
def _sc_agg(tab_h, src_h, dst_h, zeros_h, out_h, sidx, didx, rows, tabsp, acc,
            isem, gsem, ssem):
    c = lax.axis_index("c")
    s = lax.axis_index("s")
    base = c * CPB + s * CPW
    pltpu.sync_copy(tab_h.at[c].at[pl.ds(s * TSTR, TSTR)],
                    tabsp.at[pl.ds(s * TSTR, TSTR)])
    pltpu.sync_copy(zeros_h.at[pl.ds(s * RZ, RZ)], acc.at[pl.ds(s * RZ, RZ)])
    plsc.subcore_barrier()

    def idx_load(k, b):
        pltpu.async_copy(src_h.at[base + k], sidx.at[b], isem.at[b])
        pltpu.async_copy(dst_h.at[base + k], didx.at[b], isem.at[b])

    def idx_wait(k, b):
        pltpu.make_async_copy(src_h.at[base + k], sidx.at[b], isem.at[b]).wait()
        pltpu.make_async_copy(dst_h.at[base + k], didx.at[b], isem.at[b]).wait()

    for b in range(NBUF):
        idx_load(b, b)

    def round_(r, carry):
        for b in range(NBUF):
            k = r * NBUF + b
            idx_wait(k, b)
            pltpu.async_copy(tabsp.at[sidx.at[b]], rows.at[b], gsem.at[b])
        for b in range(NBUF):
            pltpu.make_async_copy(
                tabsp.at[sidx.at[b]], rows.at[b], gsem.at[b]).wait()
            pltpu.async_copy(rows.at[b], acc.at[didx.at[b]], ssem.at[b],
                             add=True)
        for b in range(NBUF):
            k = r * NBUF + b
            pltpu.make_async_copy(
                rows.at[b], acc.at[didx.at[b]], ssem.at[b]).wait()

            def refire(b=b, k=k):
                idx_load(k + NBUF, b)
            pl.when(r < ROUNDS - 1)(refire)
        return carry

    lax.fori_loop(0, ROUNDS, round_, 0)
    plsc.subcore_barrier()
    pltpu.sync_copy(acc.at[pl.ds(s * RZ, RZ)], out_h.at[c].at[pl.ds(s * RZ, RZ)])


@functools.partial(
    pl.kernel,
    out_type=jax.ShapeDtypeStruct((NC, ACC_ROWS, 16), jnp.float32),
    mesh=_mesh,
    scratch_types=[
        pltpu.VMEM((NBUF, CH), jnp.int32),
        pltpu.VMEM((CH, 16), jnp.float32),
        pltpu.VMEM_SHARED((ACC_ROWS, 16), jnp.float32),
        pltpu.SemaphoreType.DMA((NBUF,)),
        pltpu.SemaphoreType.DMA((NBUF,)),
    ],
    compiler_params=pltpu.CompilerParams(use_tc_tiling_on_sc=False),
)
def _sc_deg(dst_h, zeros_h, ones_h, out_h, didx, ones_v, acc, isem, ssem):
    c = lax.axis_index("c")
    s = lax.axis_index("s")
    base = c * CPB + s * CPW
    pltpu.sync_copy(zeros_h.at[pl.ds(s * RZ, RZ)], acc.at[pl.ds(s * RZ, RZ)])
    pltpu.sync_copy(ones_h, ones_v)
    plsc.subcore_barrier()

    def idx_load(k, b):
        pltpu.async_copy(dst_h.at[base + k], didx.at[b], isem.at[b])

    for b in range(NBUF):
        idx_load(b, b)

    def round_(r, carry):
        for b in range(NBUF):
            k = r * NBUF + b
            pltpu.make_async_copy(
                dst_h.at[base + k], didx.at[b], isem.at[b]).wait()
            pltpu.async_copy(ones_v, acc.at[didx.at[b]], ssem.at[b], add=True)
        for b in range(NBUF):
            k = r * NBUF + b
            pltpu.make_async_copy(ones_v, acc.at[didx.at[b]], ssem.at[b]).wait()

            def refire(b=b, k=k):
                idx_load(k + NBUF, b)
            pl.when(r < ROUNDS - 1)(refire)
        return carry

    lax.fori_loop(0, ROUNDS, round_, 0)
    plsc.subcore_barrier()
    pltpu.sync_copy(acc.at[pl.ds(s * RZ, RZ)], out_h.at[c].at[pl.ds(s * RZ, RZ)])


_BR = 1000
_GR = N // _BR


def _dinv_body(p_ref, o_ref):
    deg = p_ref[0, :, 0:1] + 1.0
    o_ref[0] = jnp.broadcast_to(lax.rsqrt(deg), o_ref.shape[1:])


def _dinv(parts16):
    return pl.pallas_call(
        _dinv_body,
        grid=(NC, _GR),
        in_specs=[pl.BlockSpec((1, _BR, 16), lambda c, i: (c, i, 0))],
        out_specs=pl.BlockSpec((1, _BR, D), lambda c, i: (c, i, 0)),
        out_shape=jax.ShapeDtypeStruct((NC, N, D), jnp.float32),
    )(parts16)


def _mm1_body(x_ref, w_ref, dinv_ref, o_ref):
    h = jnp.dot(x_ref[0], w_ref[0], preferred_element_type=jnp.float32)
    o_ref[0] = h * dinv_ref[0]


def _mm1(x, w, dinv):
    fin = x.shape[-1]
    return pl.pallas_call(
        _mm1_body,
        grid=(NC, _GR),
        in_specs=[
            pl.BlockSpec((1, _BR, fin), lambda c, i: (c, i, 0)),
            pl.BlockSpec((1, fin, D), lambda c, i: (c, 0, 0)),
            pl.BlockSpec((1, _BR, D), lambda c, i: (c, i, 0)),
        ],
        out_specs=pl.BlockSpec((1, _BR, D), lambda c, i: (c, i, 0)),
        out_shape=jax.ShapeDtypeStruct((NC, N, D), jnp.float32),
    )(x, w, dinv)


def _mid_body(p_ref, hs_ref, dinv_ref, b_ref, w_ref, o_ref):
    x = dinv_ref[0] * (p_ref[0] + hs_ref[0]) + b_ref[0, 0:1, :]
    x = jnp.maximum(x, 0.0)
    h = jnp.dot(x, w_ref[0], preferred_element_type=jnp.float32)
    o_ref[0] = h * dinv_ref[0]


def _mid(parts, hs, dinv, b, w):
    return pl.pallas_call(
        _mid_body,
        grid=(NC, _GR),
        in_specs=[
            pl.BlockSpec((1, _BR, D), lambda c, i: (c, i, 0)),
            pl.BlockSpec((1, _BR, D), lambda c, i: (c, i, 0)),
            pl.BlockSpec((1, _BR, D), lambda c, i: (c, i, 0)),
            pl.BlockSpec((1, 8, D), lambda c, i: (c, 0, 0)),
            pl.BlockSpec((1, D, D), lambda c, i: (c, 0, 0)),
        ],
        out_specs=pl.BlockSpec((1, _BR, D), lambda c, i: (c, i, 0)),
        out_shape=jax.ShapeDtypeStruct((NC, N, D), jnp.float32),
    )(parts, hs, dinv, b, w)


def _fin_body(p_ref, hs_ref, dinv_ref, b_ref, o_ref):
    o_ref[0] = dinv_ref[0] * (p_ref[0] + hs_ref[0]) + b_ref[0, 0:1, :]


def _fin(parts, hs, dinv, b):
    return pl.pallas_call(
        _fin_body,
        grid=(NC, _GR),
        in_specs=[
            pl.BlockSpec((1, _BR, D), lambda c, i: (c, i, 0)),
            pl.BlockSpec((1, _BR, D), lambda c, i: (c, i, 0)),
            pl.BlockSpec((1, _BR, D), lambda c, i: (c, i, 0)),
            pl.BlockSpec((1, 8, D), lambda c, i: (c, 0, 0)),
        ],
        out_specs=pl.BlockSpec((1, _BR, D), lambda c, i: (c, i, 0)),
        out_shape=jax.ShapeDtypeStruct((NC, N, D), jnp.float32),
    )(parts, hs, dinv, b)


def _pad_edges(row, fill):
    return jnp.concatenate(
        [row, jnp.full((EPADB - E,), fill, jnp.int32)]).reshape(CPB, CH)


def kernel(x_data_matrix, x_edge_index, y_data_matrix, y_edge_index,
           Wx1, bx1, Wx2, bx2, Wx3, bx3, Wx4, bx4, Wx5, bx5,
           Wy1, by1, Wy2, by2, Wy3, by3, Wy4, by4, Wy5, by5):
    src_h = jnp.concatenate([_pad_edges(x_edge_index[0], 0),
                             _pad_edges(y_edge_index[0], 0)])
    dst_h = jnp.concatenate([_pad_edges(x_edge_index[1], TRASH),
                             _pad_edges(y_edge_index[1], TRASH)])

    zeros64 = jnp.zeros((ACC_ROWS, D), jnp.float32)
    zeros16 = jnp.zeros((ACC_ROWS, 16), jnp.float32)
    ones = jnp.ones((CH, 16), jnp.float32)

    ws = [jnp.stack([Wx1, Wy1]), jnp.stack([Wx2, Wy2]), jnp.stack([Wx3, Wy3]),
          jnp.stack([Wx4, Wy4]), jnp.stack([Wx5, Wy5])]
    bs = [jnp.broadcast_to(jnp.stack([bx, by])[:, None, :], (2, 8, D))
          for bx, by in [(bx1, by1), (bx2, by2), (bx3, by3), (bx4, by4),
                         (bx5, by5)]]

    xall = jnp.stack([x_data_matrix, y_data_matrix])

    degp = _sc_deg(dst_h, zeros16, ones)
    dinv = _dinv(degp)

    hs = _mm1(xall, ws[0], dinv)
    out = None
    for li in range(5):
        parts = _sc_agg(hs, src_h, dst_h, zeros64)
        if li < 4:
            hs = _mid(parts, hs, dinv, bs[li], ws[li + 1])
        else:
            out = _fin(parts, hs, dinv, bs[4])
    return out[0], out[1]


# speedup vs baseline: 2.5835x; 1.2456x over previous
import functools

import jax
import jax.numpy as jnp
from jax import lax
from jax.experimental import pallas as pl
from jax.experimental.pallas import tpu as pltpu
from jax.experimental.pallas import tpu_sc as plsc

N = 10000
E = 320000
D = 64
NC, NS = 2, 16
CH = 128
TRASH = N
ACC_ROWS = 10240
RZ = ACC_ROWS // NS
NBUF = 4
CPW = 160
CPB = NS * CPW
EPADB = CPB * CH
ROUNDS = CPW // NBUF
TSTR = N // NS

_mesh = plsc.VectorSubcoreMesh(core_axis_name="c", subcore_axis_name="s")


@functools.partial(
    pl.kernel,
    out_type=jax.ShapeDtypeStruct((NC, ACC_ROWS, D), jnp.float32),
    mesh=_mesh,
    scratch_types=[
        pltpu.VMEM((2 * NBUF, CH), jnp.int32),
        pltpu.VMEM((2 * NBUF, CH), jnp.int32),
        pltpu.VMEM((NBUF, CH, D), jnp.float32),
        pltpu.VMEM_SHARED((ACC_ROWS, D), jnp.float32),
        pltpu.VMEM_SHARED((ACC_ROWS, D), jnp.float32),
        pltpu.SemaphoreType.DMA((2 * NBUF,)),
        pltpu.SemaphoreType.DMA((NBUF,)),
        pltpu.SemaphoreType.DMA((NBUF,)),
    ],
    compiler_params=pltpu.CompilerParams(use_tc_tiling_on_sc=False),
)
def _sc_agg(tab_h, src_h, dst_h, zeros_h, out_h, sidx, didx, rows, tabsp, acc,
            isem, gsem, ssem):
    c = lax.axis_index("c")
    s = lax.axis_index("s")
    base = c * CPB + s * CPW
    pltpu.sync_copy(tab_h.at[c].at[pl.ds(s * TSTR, TSTR)],
                    tabsp.at[pl.ds(s * TSTR, TSTR)])
    pltpu.sync_copy(zeros_h.at[pl.ds(s * RZ, RZ)], acc.at[pl.ds(s * RZ, RZ)])
    plsc.subcore_barrier()

    def idx_load(k, bank):
        pltpu.async_copy(src_h.at[base + k], sidx.at[bank], isem.at[bank])
        pltpu.async_copy(dst_h.at[base + k], didx.at[bank], isem.at[bank])

    def idx_wait(k, bank):
        pltpu.make_async_copy(
            src_h.at[base + k], sidx.at[bank], isem.at[bank]).wait()
        pltpu.make_async_copy(
            dst_h.at[base + k], didx.at[bank], isem.at[bank]).wait()

    for kb in range(2 * NBUF):
        idx_load(kb, kb)

    def half_round(r, bank_off, drain, prefetch):
        for b in range(NBUF):
            k = r * NBUF + b
            if drain is not None:
                def drain_(b=b):
                    pltpu.make_async_copy(
                        rows.at[b], acc.at[didx.at[b]], ssem.at[b]).wait()
                (pl.when(drain)(drain_) if drain is not True else drain_())
            if prefetch is not None:
                def pref_(b=b, k=k):
                    idx_load(k + NBUF, (bank_off ^ NBUF) + b)
                (pl.when(prefetch)(pref_) if prefetch is not True else pref_())
        for b in range(NBUF):
            k = r * NBUF + b
            bank = bank_off + b
            idx_wait(k, bank)
            pltpu.async_copy(tabsp.at[sidx.at[bank]], rows.at[b], gsem.at[b])
        for b in range(NBUF):
            bank = bank_off + b
            pltpu.make_async_copy(
                tabsp.at[sidx.at[bank]], rows.at[b], gsem.at[b]).wait()
            pltpu.async_copy(rows.at[b], acc.at[didx.at[bank]], ssem.at[b],
                             add=True)

    G = ROUNDS // 2

    def pair_(g, carry):
        half_round(2 * g, 0, drain=(g > 0), prefetch=(g > 0))
        half_round(2 * g + 1, NBUF, drain=True, prefetch=(g < G - 1))
        return carry

    lax.fori_loop(0, G, pair_, 0)
    for b in range(NBUF):
        pltpu.make_async_copy(rows.at[b], acc.at[didx.at[b]], ssem.at[b]).wait()
    plsc.subcore_barrier()
    pltpu.sync_copy(acc.at[pl.ds(s * RZ, RZ)], out_h.at[c].at[pl.ds(s * RZ, RZ)])


@functools.partial(
    pl.kernel,
    out_type=jax.ShapeDtypeStruct((NC, ACC_ROWS, 16), jnp.float32),
    mesh=_mesh,
    scratch_types=[
        pltpu.VMEM((NBUF, CH), jnp.int32),
        pltpu.VMEM((CH, 16), jnp.float32),
        pltpu.VMEM_SHARED((ACC_ROWS, 16), jnp.float32),
        pltpu.SemaphoreType.DMA((NBUF,)),
        pltpu.SemaphoreType.DMA((NBUF,)),
    ],
    compiler_params=pltpu.CompilerParams(use_tc_tiling_on_sc=False),
)
def _sc_deg(dst_h, zeros_h, ones_h, out_h, didx, ones_v, acc, isem, ssem):
    c = lax.axis_index("c")
    s = lax.axis_index("s")
    base = c * CPB + s * CPW
    pltpu.sync_copy(zeros_h.at[pl.ds(s * RZ, RZ)], acc.at[pl.ds(s * RZ, RZ)])
    pltpu.sync_copy(ones_h, ones_v)
    plsc.subcore_barrier()

    def idx_load(k, b):
        pltpu.async_copy(dst_h.at[base + k], didx.at[b], isem.at[b])

    for b in range(NBUF):
        idx_load(b, b)

    def round_(r, carry):
        for b in range(NBUF):
            k = r * NBUF + b
            pltpu.make_async_copy(
                dst_h.at[base + k], didx.at[b], isem.at[b]).wait()
            pltpu.async_copy(ones_v, acc.at[didx.at[b]], ssem.at[b], add=True)
        for b in range(NBUF):
            k = r * NBUF + b
            pltpu.make_async_copy(ones_v, acc.at[didx.at[b]], ssem.at[b]).wait()

            def refire(b=b, k=k):
                idx_load(k + NBUF, b)
            pl.when(r < ROUNDS - 1)(refire)
        return carry

    lax.fori_loop(0, ROUNDS, round_, 0)
    plsc.subcore_barrier()
    pltpu.sync_copy(acc.at[pl.ds(s * RZ, RZ)], out_h.at[c].at[pl.ds(s * RZ, RZ)])


_BR = 1000
_GR = N // _BR


def _dinv_scale_body(p_ref, h_ref, d_ref, hs_ref):
    deg = p_ref[0, :, 0:1] + 1.0
    dinv = jnp.broadcast_to(lax.rsqrt(deg), d_ref.shape[1:])
    d_ref[0] = dinv
    hs_ref[0] = h_ref[0] * dinv


def _dinv_scale(parts16, h1):
    return pl.pallas_call(
        _dinv_scale_body,
        grid=(NC, _GR),
        in_specs=[
            pl.BlockSpec((1, _BR, 16), lambda c, i: (c, i, 0)),
            pl.BlockSpec((1, _BR, D), lambda c, i: (c, i, 0)),
        ],
        out_specs=[
            pl.BlockSpec((1, _BR, D), lambda c, i: (c, i, 0)),
            pl.BlockSpec((1, _BR, D), lambda c, i: (c, i, 0)),
        ],
        out_shape=[
            jax.ShapeDtypeStruct((NC, N, D), jnp.float32),
            jax.ShapeDtypeStruct((NC, N, D), jnp.float32),
        ],
    )(parts16, h1)


def _mm1_body(x_ref, w_ref, o_ref):
    o_ref[0] = jnp.dot(x_ref[0], w_ref[0], preferred_element_type=jnp.float32)


def _mm1(x, w):
    fin = x.shape[-1]
    return pl.pallas_call(
        _mm1_body,
        grid=(NC, _GR),
        in_specs=[
            pl.BlockSpec((1, _BR, fin), lambda c, i: (c, i, 0)),
            pl.BlockSpec((1, fin, D), lambda c, i: (c, 0, 0)),
        ],
        out_specs=pl.BlockSpec((1, _BR, D), lambda c, i: (c, i, 0)),
        out_shape=jax.ShapeDtypeStruct((NC, N, D), jnp.float32),
    )(x, w)


def _mid_body(p_ref, hs_ref, dinv_ref, b_ref, w_ref, o_ref):
    x = dinv_ref[0] * (p_ref[0] + hs_ref[0]) + b_ref[0, 0:1, :]
    x = jnp.maximum(x, 0.0)
    h = jnp.dot(x, w_ref[0], preferred_element_type=jnp.float32)
    o_ref[0] = h * dinv_ref[0]


def _mid(parts, hs, dinv, b, w):
    return pl.pallas_call(
        _mid_body,
        grid=(NC, _GR),
        in_specs=[
            pl.BlockSpec((1, _BR, D), lambda c, i: (c, i, 0)),
            pl.BlockSpec((1, _BR, D), lambda c, i: (c, i, 0)),
            pl.BlockSpec((1, _BR, D), lambda c, i: (c, i, 0)),
            pl.BlockSpec((1, 8, D), lambda c, i: (c, 0, 0)),
            pl.BlockSpec((1, D, D), lambda c, i: (c, 0, 0)),
        ],
        out_specs=pl.BlockSpec((1, _BR, D), lambda c, i: (c, i, 0)),
        out_shape=jax.ShapeDtypeStruct((NC, N, D), jnp.float32),
    )(parts, hs, dinv, b, w)


def _fin_body(p_ref, hs_ref, dinv_ref, b_ref, o_ref):
    o_ref[0] = dinv_ref[0] * (p_ref[0] + hs_ref[0]) + b_ref[0, 0:1, :]


def _fin(parts, hs, dinv, b):
    return pl.pallas_call(
        _fin_body,
        grid=(NC, _GR),
        in_specs=[
            pl.BlockSpec((1, _BR, D), lambda c, i: (c, i, 0)),
            pl.BlockSpec((1, _BR, D), lambda c, i: (c, i, 0)),
            pl.BlockSpec((1, _BR, D), lambda c, i: (c, i, 0)),
            pl.BlockSpec((1, 8, D), lambda c, i: (c, 0, 0)),
        ],
        out_specs=pl.BlockSpec((1, _BR, D), lambda c, i: (c, i, 0)),
        out_shape=jax.ShapeDtypeStruct((NC, N, D), jnp.float32),
    )(parts, hs, dinv, b)


def _pad_edges(row, fill):
    return jnp.concatenate(
        [row, jnp.full((EPADB - E,), fill, jnp.int32)]).reshape(CPB, CH)


def kernel(x_data_matrix, x_edge_index, y_data_matrix, y_edge_index,
           Wx1, bx1, Wx2, bx2, Wx3, bx3, Wx4, bx4, Wx5, bx5,
           Wy1, by1, Wy2, by2, Wy3, by3, Wy4, by4, Wy5, by5):
    src_h = jnp.concatenate([_pad_edges(x_edge_index[0], 0),
                             _pad_edges(y_edge_index[0], 0)])
    dst_h = jnp.concatenate([_pad_edges(x_edge_index[1], TRASH),
                             _pad_edges(y_edge_index[1], TRASH)])

    zeros64 = jnp.zeros((ACC_ROWS, D), jnp.float32)
    zeros16 = jnp.zeros((ACC_ROWS, 16), jnp.float32)
    ones = jnp.ones((CH, 16), jnp.float32)

    ws = [jnp.stack([Wx1, Wy1]), jnp.stack([Wx2, Wy2]), jnp.stack([Wx3, Wy3]),
          jnp.stack([Wx4, Wy4]), jnp.stack([Wx5, Wy5])]
    bs = [jnp.broadcast_to(jnp.stack([bx, by])[:, None, :], (2, 8, D))
          for bx, by in [(bx1, by1), (bx2, by2), (bx3, by3), (bx4, by4),
                         (bx5, by5)]]

    xall = jnp.stack([x_data_matrix, y_data_matrix])

    h1 = _mm1(xall, ws[0])
    degp = _sc_deg(dst_h, zeros16, ones)
    dinv, hs = _dinv_scale(degp, h1)
    out = None
    for li in range(5):
        parts = _sc_agg(hs, src_h, dst_h, zeros64)
        if li < 4:
            hs = _mid(parts, hs, dinv, bs[li], ws[li + 1])
        else:
            out = _fin(parts, hs, dinv, bs[4])
    return out[0], out[1]
